# baseline (speedup 1.0000x reference)
import jax
import jax.numpy as jnp
from jax import lax
from jax.experimental import pallas as pl
from jax.experimental.pallas import tpu as pltpu
from jax.experimental.pallas import tpu_sc as plsc

NC = 2
NS = 16
NW = NC * NS

G = 128
CB = 8
RW = 128


def _mesh():
    return plsc.VectorSubcoreMesh(core_axis_name="c", subcore_axis_name="s",
                                  num_cores=NC, num_subcores=NS)


def _make_sc_agg(n_pad, d, e, split0):
    nch = e // G
    rpt = n_pad // NS
    rb = rpt // RW
    tot = nch // NS
    assert nch % NS == 0 and tot % CB == 0 and split0 % CB == 0
    assert rpt % RW == 0

    def body(h_hbm, src_hbm, dst_hbm, zrow_hbm, ridx_hbm, agg_out,
             src_v, dst_v, rows_v, ridx_v, agg_sp, gsem):
        cid = lax.axis_index("c")
        sid = lax.axis_index("s")

        base = sid * tot + cid * split0
        ng = (split0 + cid * (tot - 2 * split0)) // CB

        pltpu.sync_copy(ridx_hbm.at[sid], ridx_v)
        pltpu.sync_copy(zrow_hbm, rows_v.at[0])

        def iz(r, carry):
            pltpu.sync_copy(rows_v.at[0], agg_sp.at[ridx_v.at[r]])
            return carry
        lax.fori_loop(0, rb, iz, 0)

        plsc.subcore_barrier()

        pltpu.sync_copy(src_hbm.at[pl.ds(base, CB)], src_v.at[0])
        pltpu.sync_copy(dst_hbm.at[pl.ds(base, CB)], dst_v.at[0])
        pltpu.async_copy(h_hbm.at[src_v.at[0, 0]], rows_v.at[0], gsem)

        def group(gi, carry):
            slot = lax.rem(gi, 2)
            nslot = lax.rem(gi + 1, 2)

            @pl.when(gi + 1 < ng)
            def _stage_next():
                pltpu.sync_copy(
                    src_hbm.at[pl.ds(base + (gi + 1) * CB, CB)],
                    src_v.at[nslot])
                pltpu.sync_copy(
                    dst_hbm.at[pl.ds(base + (gi + 1) * CB, CB)],
                    dst_v.at[nslot])

            for k in range(CB):
                pltpu.make_async_copy(h_hbm.at[src_v.at[slot, k]],
                                      rows_v.at[k % 2], gsem).wait()
                if k + 1 < CB:
                    pltpu.async_copy(h_hbm.at[src_v.at[slot, k + 1]],
                                     rows_v.at[(k + 1) % 2], gsem)
                else:
                    @pl.when(gi + 1 < ng)
                    def _issue_next_group():
                        pltpu.async_copy(h_hbm.at[src_v.at[nslot, 0]],
                                         rows_v.at[0], gsem)
                pltpu.sync_copy(rows_v.at[k % 2],
                                agg_sp.at[dst_v.at[slot, k]], add=True)
            return carry

        lax.fori_loop(0, ng, group, 0)

        plsc.subcore_barrier()

        def cout(r, carry):
            pltpu.sync_copy(agg_sp.at[ridx_v.at[r]], rows_v.at[0])
            pltpu.sync_copy(rows_v.at[0],
                            agg_out.at[cid, pl.ds(sid * rpt + r * RW, RW)])
            return carry
        lax.fori_loop(0, rb, cout, 0)

    return pl.kernel(
        body,
        out_type=[jax.ShapeDtypeStruct((NC, n_pad, d), jnp.float32)],
        mesh=_mesh(),
        scratch_types=[
            pltpu.VMEM((2, CB, G), jnp.int32),
            pltpu.VMEM((2, CB, G), jnp.int32),
            pltpu.VMEM((2, G, d), jnp.float32),
            pltpu.VMEM((rpt // RW, RW), jnp.int32),
            pltpu.VMEM_SHARED((n_pad, d), jnp.float32),
            pltpu.SemaphoreType.DMA,
        ])


def _make_sc_deg(n_pad, d, e):
    c = e // (NW * G)
    rpt = n_pad // NS
    rb = rpt // RW
    assert c % CB == 0 and rpt % RW == 0

    def body(dst_hbm, ones_hbm, zrow_hbm, ridx_hbm, deg_out,
             dst_v, ones_v, zrow_v, ridx_v, deg_sp):
        cid = lax.axis_index("c")
        sid = lax.axis_index("s")
        wid = cid * NS + sid

        pltpu.sync_copy(ones_hbm, ones_v)
        pltpu.sync_copy(zrow_hbm, zrow_v)
        pltpu.sync_copy(ridx_hbm.at[sid], ridx_v)

        def iz(r, carry):
            pltpu.sync_copy(zrow_v, deg_sp.at[ridx_v.at[r]])
            return carry
        lax.fori_loop(0, rb, iz, 0)

        plsc.subcore_barrier()

        def outer(ci, carry):
            pltpu.sync_copy(dst_hbm.at[wid, pl.ds(ci * CB, CB)], dst_v)

            def inner(j, carry2):
                pltpu.sync_copy(ones_v, deg_sp.at[dst_v.at[j]], add=True)
                return carry2

            lax.fori_loop(0, CB, inner, 0)
            return carry

        lax.fori_loop(0, c // CB, outer, 0)

        plsc.subcore_barrier()

        def cout(r, carry):
            pltpu.sync_copy(deg_sp.at[ridx_v.at[r]], zrow_v)
            pltpu.sync_copy(zrow_v,
                            deg_out.at[cid, pl.ds(sid * rpt + r * RW, RW)])
            return carry
        lax.fori_loop(0, rb, cout, 0)

    return pl.kernel(
        body,
        out_type=[jax.ShapeDtypeStruct((NC, n_pad, d), jnp.float32)],
        mesh=_mesh(),
        scratch_types=[
            pltpu.VMEM((CB, G), jnp.int32),
            pltpu.VMEM((G, d), jnp.float32),
            pltpu.VMEM((RW, d), jnp.float32),
            pltpu.VMEM((rpt // RW, RW), jnp.int32),
            pltpu.VMEM_SHARED((n_pad, d), jnp.float32),
        ])


def _tc_layer1_body(x_ref, aggp_ref, degp_ref, ws_ref, wn_ref, b_ref, o_ref):
    agg = aggp_ref[0] + aggp_ref[1]
    deg = degp_ref[0] + degp_ref[1]
    inv = 1.0 / jnp.maximum(deg[:, 0:1], 1.0)
    h = (x_ref[...] @ ws_ref[...] + (agg * inv) @ wn_ref[...] + b_ref[...])
    o_ref[...] = jnp.maximum(h, 0.0)


def _tc_layer2_body(h_ref, aggp_ref, degp_ref, ws_ref, wn_ref, b_ref,
                    wh_ref, bh_ref, o_ref):
    agg = aggp_ref[0] + aggp_ref[1]
    deg = degp_ref[0] + degp_ref[1]
    inv = 1.0 / jnp.maximum(deg[:, 0:1], 1.0)
    h2 = jnp.maximum(
        h_ref[...] @ ws_ref[...] + (agg * inv) @ wn_ref[...] + b_ref[...],
        0.0)
    o_ref[...] = h2 @ wh_ref[...] + bh_ref[...]


def _make_tc_layer1(n, d, blk):
    grid = (n // blk,)
    return pl.pallas_call(
        _tc_layer1_body,
        grid=grid,
        in_specs=[
            pl.BlockSpec((blk, d), lambda i: (i, 0)),
            pl.BlockSpec((NC, blk, d), lambda i: (0, i, 0)),
            pl.BlockSpec((NC, blk, d), lambda i: (0, i, 0)),
            pl.BlockSpec((d, d), lambda i: (0, 0)),
            pl.BlockSpec((d, d), lambda i: (0, 0)),
            pl.BlockSpec((1, d), lambda i: (0, 0)),
        ],
        out_specs=pl.BlockSpec((blk, d), lambda i: (i, 0)),
        out_shape=jax.ShapeDtypeStruct((n, d), jnp.float32),
    )


def _make_tc_layer2(n, d, out_d, blk):
    grid = (n // blk,)
    return pl.pallas_call(
        _tc_layer2_body,
        grid=grid,
        in_specs=[
            pl.BlockSpec((blk, d), lambda i: (i, 0)),
            pl.BlockSpec((NC, blk, d), lambda i: (0, i, 0)),
            pl.BlockSpec((NC, blk, d), lambda i: (0, i, 0)),
            pl.BlockSpec((d, d), lambda i: (0, 0)),
            pl.BlockSpec((d, d), lambda i: (0, 0)),
            pl.BlockSpec((1, d), lambda i: (0, 0)),
            pl.BlockSpec((d, out_d), lambda i: (0, 0)),
            pl.BlockSpec((1, out_d), lambda i: (0, 0)),
        ],
        out_specs=pl.BlockSpec((blk, out_d), lambda i: (i, 0)),
        out_shape=jax.ShapeDtypeStruct((n, out_d), jnp.float32),
    )


def kernel(x, edge_index, W_self1, W_nbr1, b1, W_self2, W_nbr2, b2, W_head,
           b_head):
    n, d = x.shape
    e = edge_index.shape[1]
    out_d = W_head.shape[1]
    n_pad = ((n + NS * RW - 1) // (NS * RW)) * (NS * RW)
    rpt = n_pad // NS
    grp = NW * G * CB
    e_pad = ((e + grp - 1) // grp) * grp
    c = e_pad // (NW * G)

    src = jnp.concatenate(
        [edge_index[0], jnp.zeros((e_pad - e,), jnp.int32)]).reshape(NW, c, G)
    dst = jnp.concatenate(
        [edge_index[1],
         jnp.full((e_pad - e,), n_pad - 1, jnp.int32)]).reshape(NW, c, G)
    ones_g = jnp.ones((G, d), jnp.float32)
    zrow_d = jnp.zeros((RW, d), jnp.float32)
    ridx = jnp.arange(n_pad, dtype=jnp.int32).reshape(NS, rpt // RW, RW)

    src2 = src.reshape(e_pad // G, G)
    dst2 = dst.reshape(e_pad // G, G)

    sc_agg = _make_sc_agg(n_pad, d, e_pad, split0=40)
    sc_deg = _make_sc_deg(n_pad, d, e_pad)
    blk = 400
    tc1 = _make_tc_layer1(n, d, blk)
    tc2 = _make_tc_layer2(n, d, out_d, blk)

    (degp,) = sc_deg(dst, ones_g, zrow_d, ridx)
    (aggp1,) = sc_agg(x, src2, dst2, zrow_d, ridx)
    h1 = tc1(x, aggp1, degp, W_self1, W_nbr1, b1.reshape(1, d))
    (aggp2,) = sc_agg(h1, src2, dst2, zrow_d, ridx)
    out = tc2(h1, aggp2, degp, W_self2, W_nbr2, b2.reshape(1, d),
              W_head, b_head.reshape(1, out_d))
    return out


# speedup vs baseline: 1.0688x; 1.0688x over previous
import jax
import jax.numpy as jnp
from jax import lax
from jax.experimental import pallas as pl
from jax.experimental.pallas import tpu as pltpu
from jax.experimental.pallas import tpu_sc as plsc

NC = 2
NS = 16
NW = NC * NS

G = 128
CB = 8
RW = 128


def _mesh():
    return plsc.VectorSubcoreMesh(core_axis_name="c", subcore_axis_name="s",
                                  num_cores=NC, num_subcores=NS)


def _make_sc_agg(n_pad, d, e, split0):
    nch = e // G
    rpt = n_pad // NS
    rb = rpt // RW
    tot = nch // NS
    assert nch % NS == 0 and tot % CB == 0 and split0 % CB == 0
    assert rpt % RW == 0

    def body(h_hbm, src_hbm, dst_hbm, zrow_hbm, ridx_hbm, agg_out,
             src_v, dst_v, rows_v, ridx_v, agg_sp, gsem):
        cid = lax.axis_index("c")
        sid = lax.axis_index("s")

        base = sid * tot + cid * split0
        ng = (split0 + cid * (tot - 2 * split0)) // CB

        pltpu.sync_copy(ridx_hbm.at[sid], ridx_v)
        pltpu.sync_copy(zrow_hbm, rows_v.at[0])

        def iz(r, carry):
            pltpu.sync_copy(rows_v.at[0], agg_sp.at[ridx_v.at[r]])
            return carry
        lax.fori_loop(0, rb, iz, 0)

        plsc.subcore_barrier()

        pltpu.sync_copy(src_hbm.at[pl.ds(base, CB)], src_v.at[0])
        pltpu.sync_copy(dst_hbm.at[pl.ds(base, CB)], dst_v.at[0])
        pltpu.async_copy(h_hbm.at[src_v.at[0, 0]], rows_v.at[0], gsem)

        def group(gi, carry):
            slot = lax.rem(gi, 2)
            nslot = lax.rem(gi + 1, 2)

            @pl.when(gi + 1 < ng)
            def _stage_next():
                pltpu.sync_copy(
                    src_hbm.at[pl.ds(base + (gi + 1) * CB, CB)],
                    src_v.at[nslot])
                pltpu.sync_copy(
                    dst_hbm.at[pl.ds(base + (gi + 1) * CB, CB)],
                    dst_v.at[nslot])

            for k in range(CB):
                pltpu.make_async_copy(h_hbm.at[src_v.at[slot, k]],
                                      rows_v.at[k % 2], gsem).wait()
                if k + 1 < CB:
                    pltpu.async_copy(h_hbm.at[src_v.at[slot, k + 1]],
                                     rows_v.at[(k + 1) % 2], gsem)
                else:
                    @pl.when(gi + 1 < ng)
                    def _issue_next_group():
                        pltpu.async_copy(h_hbm.at[src_v.at[nslot, 0]],
                                         rows_v.at[0], gsem)
                pltpu.sync_copy(rows_v.at[k % 2],
                                agg_sp.at[dst_v.at[slot, k]], add=True)
            return carry

        lax.fori_loop(0, ng, group, 0)

        plsc.subcore_barrier()

        def cout(r, carry):
            pltpu.sync_copy(agg_sp.at[ridx_v.at[r]], rows_v.at[0])
            pltpu.sync_copy(rows_v.at[0],
                            agg_out.at[cid, pl.ds(sid * rpt + r * RW, RW)])
            return carry
        lax.fori_loop(0, rb, cout, 0)

    return pl.kernel(
        body,
        out_type=[jax.ShapeDtypeStruct((NC, n_pad, d), jnp.float32)],
        mesh=_mesh(),
        scratch_types=[
            pltpu.VMEM((2, CB, G), jnp.int32),
            pltpu.VMEM((2, CB, G), jnp.int32),
            pltpu.VMEM((2, G, d), jnp.float32),
            pltpu.VMEM((rpt // RW, RW), jnp.int32),
            pltpu.VMEM_SHARED((n_pad, d), jnp.float32),
            pltpu.SemaphoreType.DMA,
        ])


def _make_sc_deg(n_pad, d, e):
    c = e // (NW * G)
    rpt = n_pad // NS
    rb = rpt // RW
    assert c % CB == 0 and rpt % RW == 0

    def body(dst_hbm, ones_hbm, zrow_hbm, ridx_hbm, deg_out,
             dst_v, ones_v, zrow_v, ridx_v, deg_sp):
        cid = lax.axis_index("c")
        sid = lax.axis_index("s")
        wid = cid * NS + sid

        pltpu.sync_copy(ones_hbm, ones_v)
        pltpu.sync_copy(zrow_hbm, zrow_v)
        pltpu.sync_copy(ridx_hbm.at[sid], ridx_v)

        def iz(r, carry):
            pltpu.sync_copy(zrow_v, deg_sp.at[ridx_v.at[r]])
            return carry
        lax.fori_loop(0, rb, iz, 0)

        plsc.subcore_barrier()

        def outer(ci, carry):
            pltpu.sync_copy(dst_hbm.at[wid, pl.ds(ci * CB, CB)], dst_v)

            def inner(j, carry2):
                pltpu.sync_copy(ones_v, deg_sp.at[dst_v.at[j]], add=True)
                return carry2

            lax.fori_loop(0, CB, inner, 0)
            return carry

        lax.fori_loop(0, c // CB, outer, 0)

        plsc.subcore_barrier()

        def cout(r, carry):
            pltpu.sync_copy(deg_sp.at[ridx_v.at[r]], zrow_v)
            pltpu.sync_copy(zrow_v,
                            deg_out.at[cid, pl.ds(sid * rpt + r * RW, RW)])
            return carry
        lax.fori_loop(0, rb, cout, 0)

    return pl.kernel(
        body,
        out_type=[jax.ShapeDtypeStruct((NC, n_pad, d), jnp.float32)],
        mesh=_mesh(),
        scratch_types=[
            pltpu.VMEM((CB, G), jnp.int32),
            pltpu.VMEM((G, d), jnp.float32),
            pltpu.VMEM((RW, d), jnp.float32),
            pltpu.VMEM((rpt // RW, RW), jnp.int32),
            pltpu.VMEM_SHARED((n_pad, d), jnp.float32),
        ])


def _tc_layer1_body(x_ref, aggp_ref, degp_ref, ws_ref, wn_ref, b_ref, o_ref):
    agg = aggp_ref[0] + aggp_ref[1]
    deg = degp_ref[0] + degp_ref[1]
    inv = 1.0 / jnp.maximum(deg[:, 0:1], 1.0)
    h = (x_ref[...] @ ws_ref[...] + (agg * inv) @ wn_ref[...] + b_ref[...])
    o_ref[...] = jnp.maximum(h, 0.0)


def _tc_layer2_body(h_ref, aggp_ref, degp_ref, ws_ref, wn_ref, b_ref,
                    wh_ref, bh_ref, o_ref):
    agg = aggp_ref[0] + aggp_ref[1]
    deg = degp_ref[0] + degp_ref[1]
    inv = 1.0 / jnp.maximum(deg[:, 0:1], 1.0)
    h2 = jnp.maximum(
        h_ref[...] @ ws_ref[...] + (agg * inv) @ wn_ref[...] + b_ref[...],
        0.0)
    o_ref[...] = h2 @ wh_ref[...] + bh_ref[...]


def _make_tc_layer1(n, d, blk):
    grid = (n // blk,)
    return pl.pallas_call(
        _tc_layer1_body,
        grid=grid,
        in_specs=[
            pl.BlockSpec((blk, d), lambda i: (i, 0)),
            pl.BlockSpec((NC, blk, d), lambda i: (0, i, 0)),
            pl.BlockSpec((NC, blk, d), lambda i: (0, i, 0)),
            pl.BlockSpec((d, d), lambda i: (0, 0)),
            pl.BlockSpec((d, d), lambda i: (0, 0)),
            pl.BlockSpec((1, d), lambda i: (0, 0)),
        ],
        out_specs=pl.BlockSpec((blk, d), lambda i: (i, 0)),
        out_shape=jax.ShapeDtypeStruct((n, d), jnp.float32),
    )


def _make_tc_layer2(n, d, out_d, blk):
    grid = (n // blk,)
    return pl.pallas_call(
        _tc_layer2_body,
        grid=grid,
        in_specs=[
            pl.BlockSpec((blk, d), lambda i: (i, 0)),
            pl.BlockSpec((NC, blk, d), lambda i: (0, i, 0)),
            pl.BlockSpec((NC, blk, d), lambda i: (0, i, 0)),
            pl.BlockSpec((d, d), lambda i: (0, 0)),
            pl.BlockSpec((d, d), lambda i: (0, 0)),
            pl.BlockSpec((1, d), lambda i: (0, 0)),
            pl.BlockSpec((d, out_d), lambda i: (0, 0)),
            pl.BlockSpec((1, out_d), lambda i: (0, 0)),
        ],
        out_specs=pl.BlockSpec((blk, out_d), lambda i: (i, 0)),
        out_shape=jax.ShapeDtypeStruct((n, out_d), jnp.float32),
    )


def kernel(x, edge_index, W_self1, W_nbr1, b1, W_self2, W_nbr2, b2, W_head,
           b_head):
    n, d = x.shape
    e = edge_index.shape[1]
    out_d = W_head.shape[1]
    n_pad = ((n + NS * RW - 1) // (NS * RW)) * (NS * RW)
    rpt = n_pad // NS
    grp = NW * G * CB
    e_pad = ((e + grp - 1) // grp) * grp
    c = e_pad // (NW * G)

    src = jnp.concatenate(
        [edge_index[0], jnp.zeros((e_pad - e,), jnp.int32)]).reshape(NW, c, G)
    dst = jnp.concatenate(
        [edge_index[1],
         jnp.full((e_pad - e,), n_pad - 1, jnp.int32)]).reshape(NW, c, G)
    ones_g = jnp.ones((G, d), jnp.float32)
    zrow_d = jnp.zeros((RW, d), jnp.float32)
    ridx = jnp.arange(n_pad, dtype=jnp.int32).reshape(NS, rpt // RW, RW)

    src2 = src.reshape(e_pad // G, G)
    dst2 = dst.reshape(e_pad // G, G)

    sc_agg = _make_sc_agg(n_pad, d, e_pad, split0=80)
    sc_deg = _make_sc_deg(n_pad, d, e_pad)
    blk = 400
    tc1 = _make_tc_layer1(n, d, blk)
    tc2 = _make_tc_layer2(n, d, out_d, blk)

    (degp,) = sc_deg(dst, ones_g, zrow_d, ridx)
    (aggp1,) = sc_agg(x, src2, dst2, zrow_d, ridx)
    h1 = tc1(x, aggp1, degp, W_self1, W_nbr1, b1.reshape(1, d))
    (aggp2,) = sc_agg(h1, src2, dst2, zrow_d, ridx)
    out = tc2(h1, aggp2, degp, W_self2, W_nbr2, b2.reshape(1, d),
              W_head, b_head.reshape(1, out_d))
    return out
